# baseline (speedup 1.0000x reference)
import functools

import jax
import jax.numpy as jnp
from jax import lax
from jax.experimental import pallas as pl
from jax.experimental.pallas import tpu as pltpu
from jax.experimental.pallas import tpu_sc as plsc

D = 32
K = 128
B = 1000000
L = 16
NC = 2
NS = 16
NW = NC * NS
C = 1600
NCHUNK = B // C
ITERS = -(-NCHUNK // NW)


def _prep_body(hT_ref, x_ref, T_ref):
    hT = hT_ref[...]
    x = x_ref[...]
    yT = jax.nn.softplus(hT)
    dx = x[1:] - x[:-1]
    dyT = yT[1:, :] - yT[:-1, :]
    sT = dyT / dx[:, None]
    sT = jnp.concatenate([sT, jnp.zeros((1, D), jnp.float32)], axis=0)
    aT = yT - sT * x[:, None]
    T_ref[...] = jnp.concatenate([aT, sT], axis=1)


_prep = pl.pallas_call(
    _prep_body,
    out_shape=jax.ShapeDtypeStruct((K, 2 * D), jnp.float32),
)


def _sc_body(t_hbm, tab_hbm, x_hbm, out_hbm, t_v, tab_v, x_v, out_v0, out_v1, kb_v, sem0, sem1):
    wid = lax.axis_index("s") * NC + lax.axis_index("c")

    pltpu.sync_copy(tab_hbm, tab_v)
    pltpu.sync_copy(x_hbm, x_v)

    lanes_d = lax.iota(jnp.int32, L) * D
    ones_i = jnp.full((L,), 1, jnp.int32)
    x1 = plsc.load_gather(x_v, [ones_i])
    x2 = plsc.load_gather(x_v, [ones_i + 1])
    dx = x2 - x1
    inv_dx = 1.0 / dx
    x0 = x1 - dx

    def do_chunk(g, out_v, sem):
        chunk = g * NW + wid

        @pl.when(chunk < NCHUNK)
        def _():
            base = chunk * C
            with jax.named_scope("tload"):
                pltpu.sync_copy(t_hbm.at[pl.ds(base, C)], t_v)

            @pl.when(g >= 2)
            def _():
                with jax.named_scope("dwait"):
                    pltpu.make_async_copy(
                        out_v, out_hbm.at[pl.ds(base * D, C * D)], sem
                    ).wait()

            scopeA = jax.named_scope("passA")
            scopeA.__enter__()

            @plsc.parallel_loop(0, C // L, unroll=4)
            def index_body(i):
                tv = t_v[pl.ds(i * L, L)]
                k0 = ((tv - x0) * inv_dx).astype(jnp.int32)
                k0 = jnp.clip(k0, 0, K - 2)
                xr = plsc.load_gather(x_v, [k0 + 1])
                k1 = k0 + (tv >= xr).astype(jnp.int32)
                xl = plsc.load_gather(x_v, [k1])
                k = k1 - (tv < xl).astype(jnp.int32)
                kb_v[pl.ds(i * L, L)] = k * (2 * D)

            scopeA.__exit__(None, None, None)
            scopeB = jax.named_scope("passB")
            scopeB.__enter__()
            JG = 8
            for jg in range(0, D, JG):

                @plsc.parallel_loop(0, C // L, unroll=2)
                def gather_body(i, jg=jg):
                    tb = kb_v[pl.ds(i * L, L)]
                    tv = t_v[pl.ds(i * L, L)]
                    ob = i * (L * D) + lanes_d
                    for j in range(jg, jg + JG):
                        a = plsc.load_gather(tab_v, [tb + j])
                        s = plsc.load_gather(tab_v, [tb + (D + j)])
                        plsc.store_scatter(out_v, [ob + j], a + tv * s)
            scopeB.__exit__(None, None, None)
            pltpu.async_copy(out_v, out_hbm.at[pl.ds(base * D, C * D)], sem)

    def chunk_pair(g2, carry):
        do_chunk(g2 * 2, out_v0, sem0)
        do_chunk(g2 * 2 + 1, out_v1, sem1)
        return carry

    lax.fori_loop(0, ITERS // 2, chunk_pair, 0)
    pltpu.make_async_copy(out_v0, out_hbm.at[pl.ds(0, C * D)], sem0).wait()
    pltpu.make_async_copy(out_v1, out_hbm.at[pl.ds(0, C * D)], sem1).wait()


_sc_eval = functools.partial(
    pl.kernel,
    out_type=jax.ShapeDtypeStruct((B * D,), jnp.float32),
    mesh=plsc.VectorSubcoreMesh(
        core_axis_name="c", subcore_axis_name="s", num_cores=NC, num_subcores=NS
    ),
    scratch_types=[
        pltpu.VMEM((C,), jnp.float32),
        pltpu.VMEM((K * 2 * D,), jnp.float32),
        pltpu.VMEM((K,), jnp.float32),
        pltpu.VMEM((C * D,), jnp.float32),
        pltpu.VMEM((C * D,), jnp.float32),
        pltpu.VMEM((C,), jnp.int32),
        pltpu.SemaphoreType.DMA,
        pltpu.SemaphoreType.DMA,
    ],
    compiler_params=pltpu.CompilerParams(needs_layout_passes=False),
)(_sc_body)


def kernel(t, h_knots, x_knots):
    tab = _prep(h_knots.T, x_knots).reshape(-1)
    out_flat = _sc_eval(t, tab, x_knots)
    return out_flat.reshape(B, D)


# speedup vs baseline: 1.6247x; 1.6247x over previous
import functools

import jax
import jax.numpy as jnp
from jax import lax
from jax.experimental import pallas as pl
from jax.experimental.pallas import tpu as pltpu
from jax.experimental.pallas import tpu_sc as plsc

D = 32
K = 128
B = 1000000
L = 16
NC = 2
NS = 16
NW = NC * NS
C = 1600
NCHUNK = B // C
ITERS = -(-NCHUNK // NW)


def _prep_body(hT_ref, x_ref, T_ref):
    hT = hT_ref[...]
    x = x_ref[...]
    yT = jax.nn.softplus(hT)
    dx = x[1:] - x[:-1]
    dyT = yT[1:, :] - yT[:-1, :]
    sT = dyT / dx[:, None]
    sT = jnp.concatenate([sT, jnp.zeros((1, D), jnp.float32)], axis=0)
    aT = yT - sT * x[:, None]
    T_ref[...] = jnp.concatenate(
        [aT, jnp.zeros((K, 1), jnp.float32), sT], axis=1
    )


TW = 2 * D + 1
OW = D + 1

_prep = pl.pallas_call(
    _prep_body,
    out_shape=jax.ShapeDtypeStruct((K, TW), jnp.float32),
)


def _sc_body(t_hbm, tab_hbm, x_hbm, out_hbm, t_v, tab_v, x_v, out_v0, out_v1, kb_v, sem0, sem1):
    wid = lax.axis_index("s") * NC + lax.axis_index("c")

    pltpu.sync_copy(tab_hbm, tab_v)
    pltpu.sync_copy(x_hbm, x_v)

    lanes_d = lax.iota(jnp.int32, L) * OW
    ones_i = jnp.full((L,), 1, jnp.int32)
    x1 = plsc.load_gather(x_v, [ones_i])
    x2 = plsc.load_gather(x_v, [ones_i + 1])
    dx = x2 - x1
    inv_dx = 1.0 / dx
    x0 = x1 - dx

    def do_chunk(g, out_v, sem):
        chunk = g * NW + wid

        @pl.when(chunk < NCHUNK)
        def _():
            base = chunk * C
            with jax.named_scope("tload"):
                pltpu.sync_copy(t_hbm.at[pl.ds(base, C)], t_v)

            @pl.when(g >= 2)
            def _():
                with jax.named_scope("dwait"):
                    pltpu.make_async_copy(
                        out_v, out_hbm.at[pl.ds(base * OW, C * OW)], sem
                    ).wait()

            scopeA = jax.named_scope("passA")
            scopeA.__enter__()

            @plsc.parallel_loop(0, C // L, unroll=4)
            def index_body(i):
                tv = t_v[pl.ds(i * L, L)]
                k0 = ((tv - x0) * inv_dx).astype(jnp.int32)
                k0 = jnp.clip(k0, 0, K - 2)
                xr = plsc.load_gather(x_v, [k0 + 1])
                k1 = k0 + (tv >= xr).astype(jnp.int32)
                xl = plsc.load_gather(x_v, [k1])
                k = k1 - (tv < xl).astype(jnp.int32)
                kb_v[pl.ds(i * L, L)] = k * TW

            scopeA.__exit__(None, None, None)
            scopeB = jax.named_scope("passB")
            scopeB.__enter__()
            JG = 8
            for jg in range(0, D, JG):

                @plsc.parallel_loop(0, C // L, unroll=2)
                def gather_body(i, jg=jg):
                    tb = kb_v[pl.ds(i * L, L)]
                    tv = t_v[pl.ds(i * L, L)]
                    ob = i * (L * OW) + lanes_d
                    for j in range(jg, jg + JG):
                        a = plsc.load_gather(tab_v, [tb + j])
                        s = plsc.load_gather(tab_v, [tb + (D + 1 + j)])
                        plsc.store_scatter(out_v, [ob + j], a + tv * s)
            scopeB.__exit__(None, None, None)
            pltpu.async_copy(out_v, out_hbm.at[pl.ds(base * OW, C * OW)], sem)

    def chunk_pair(g2, carry):
        do_chunk(g2 * 2, out_v0, sem0)
        do_chunk(g2 * 2 + 1, out_v1, sem1)
        return carry

    lax.fori_loop(0, ITERS // 2, chunk_pair, 0)
    pltpu.make_async_copy(out_v0, out_hbm.at[pl.ds(0, C * OW)], sem0).wait()
    pltpu.make_async_copy(out_v1, out_hbm.at[pl.ds(0, C * OW)], sem1).wait()


_sc_eval = functools.partial(
    pl.kernel,
    out_type=jax.ShapeDtypeStruct((B * OW,), jnp.float32),
    mesh=plsc.VectorSubcoreMesh(
        core_axis_name="c", subcore_axis_name="s", num_cores=NC, num_subcores=NS
    ),
    scratch_types=[
        pltpu.VMEM((C,), jnp.float32),
        pltpu.VMEM((K * TW,), jnp.float32),
        pltpu.VMEM((K,), jnp.float32),
        pltpu.VMEM((C * OW,), jnp.float32),
        pltpu.VMEM((C * OW,), jnp.float32),
        pltpu.VMEM((C,), jnp.int32),
        pltpu.SemaphoreType.DMA,
        pltpu.SemaphoreType.DMA,
    ],
    compiler_params=pltpu.CompilerParams(needs_layout_passes=False),
)(_sc_body)


def kernel(t, h_knots, x_knots):
    tab = _prep(h_knots.T, x_knots).reshape(-1)
    out_flat = _sc_eval(t, tab, x_knots)
    return out_flat.reshape(B, OW)[:, :D]


# speedup vs baseline: 2.4061x; 1.4810x over previous
import functools

import jax
import jax.numpy as jnp
from jax import lax
from jax.experimental import pallas as pl
from jax.experimental.pallas import tpu as pltpu
from jax.experimental.pallas import tpu_sc as plsc

D = 32
K = 128
B = 1000000
L = 16
NC = 2
NS = 16
NW = NC * NS
C = 1600
NCHUNK = B // C
ITERS = -(-NCHUNK // NW)


def _prep_body(hT_ref, x_ref, T_ref):
    hT = hT_ref[...]
    x = x_ref[...]
    yT = jax.nn.softplus(hT)
    dx = x[1:] - x[:-1]
    dyT = yT[1:, :] - yT[:-1, :]
    sT = dyT / dx[:, None]
    sT = jnp.concatenate([sT, jnp.zeros((1, D), jnp.float32)], axis=0)
    aT = yT - sT * x[:, None]
    T_ref[...] = jnp.concatenate([aT, sT], axis=1)


TW = 2 * D

_prep = pl.pallas_call(
    _prep_body,
    out_shape=jax.ShapeDtypeStruct((K, TW), jnp.float32),
)


def _sc_body(t_hbm, tab_hbm, x_hbm, out_hbm, t_v, tab_v, x_v, out_v0, out_v1, kb_v, sem0, sem1):
    wid = lax.axis_index("s") * NC + lax.axis_index("c")

    pltpu.sync_copy(tab_hbm, tab_v)
    pltpu.sync_copy(x_hbm, x_v)

    lanes = lax.iota(jnp.int32, L)
    lanes_d = lanes * D
    ones_i = jnp.full((L,), 1, jnp.int32)
    x1 = plsc.load_gather(x_v, [ones_i])
    x2 = plsc.load_gather(x_v, [ones_i + 1])
    dx = x2 - x1
    inv_dx = 1.0 / dx
    x0 = x1 - dx

    def do_chunk(g, out_v, sem):
        chunk = g * NW + wid

        @pl.when(chunk < NCHUNK)
        def _():
            base = chunk * C
            with jax.named_scope("tload"):
                pltpu.sync_copy(t_hbm.at[pl.ds(base, C)], t_v)

            @pl.when(g >= 2)
            def _():
                with jax.named_scope("dwait"):
                    pltpu.make_async_copy(
                        out_v, out_hbm.at[pl.ds(base * D, C * D)], sem
                    ).wait()

            scopeA = jax.named_scope("passA")
            scopeA.__enter__()

            @plsc.parallel_loop(0, C // L, unroll=4)
            def index_body(i):
                tv = t_v[pl.ds(i * L, L)]
                k0 = ((tv - x0) * inv_dx).astype(jnp.int32)
                k0 = jnp.clip(k0, 0, K - 2)
                xr = plsc.load_gather(x_v, [k0 + 1])
                k1 = k0 + (tv >= xr).astype(jnp.int32)
                xl = plsc.load_gather(x_v, [k1])
                k = k1 - (tv < xl).astype(jnp.int32)
                kb_v[pl.ds(i * L, L)] = k * TW

            scopeA.__exit__(None, None, None)
            scopeB = jax.named_scope("passB")
            scopeB.__enter__()
            JG = 8
            for jg in range(0, D, JG):

                @plsc.parallel_loop(0, C // L, unroll=2)
                def gather_body(i, jg=jg):
                    tb = kb_v[pl.ds(i * L, L)]
                    tv = t_v[pl.ds(i * L, L)]
                    ob = i * (L * D) + lanes_d
                    for j in range(jg, jg + JG):
                        jr = (lanes + j) & (D - 1)
                        ai = tb + jr
                        a = plsc.load_gather(tab_v, [ai])
                        s = plsc.load_gather(tab_v, [ai + D])
                        plsc.store_scatter(out_v, [ob + jr], a + tv * s)
            scopeB.__exit__(None, None, None)
            pltpu.async_copy(out_v, out_hbm.at[pl.ds(base * D, C * D)], sem)

    def chunk_pair(g2, carry):
        do_chunk(g2 * 2, out_v0, sem0)
        do_chunk(g2 * 2 + 1, out_v1, sem1)
        return carry

    lax.fori_loop(0, ITERS // 2, chunk_pair, 0)
    pltpu.make_async_copy(out_v0, out_hbm.at[pl.ds(0, C * D)], sem0).wait()
    pltpu.make_async_copy(out_v1, out_hbm.at[pl.ds(0, C * D)], sem1).wait()


_sc_eval = functools.partial(
    pl.kernel,
    out_type=jax.ShapeDtypeStruct((B * D,), jnp.float32),
    mesh=plsc.VectorSubcoreMesh(
        core_axis_name="c", subcore_axis_name="s", num_cores=NC, num_subcores=NS
    ),
    scratch_types=[
        pltpu.VMEM((C,), jnp.float32),
        pltpu.VMEM((K * TW,), jnp.float32),
        pltpu.VMEM((K,), jnp.float32),
        pltpu.VMEM((C * D,), jnp.float32),
        pltpu.VMEM((C * D,), jnp.float32),
        pltpu.VMEM((C,), jnp.int32),
        pltpu.SemaphoreType.DMA,
        pltpu.SemaphoreType.DMA,
    ],
    compiler_params=pltpu.CompilerParams(needs_layout_passes=False),
)(_sc_body)


def kernel(t, h_knots, x_knots):
    tab = _prep(h_knots.T, x_knots).reshape(-1)
    out_flat = _sc_eval(t, tab, x_knots)
    return out_flat.reshape(B, D)
